# baseline (speedup 1.0000x reference)
import functools

import jax
import jax.numpy as jnp
from jax import lax
from jax.experimental import pallas as pl
from jax.experimental.pallas import tpu as pltpu
from jax.experimental.pallas import tpu_sc as plsc

N = 10000
E = 320000
IN_FEA = 128
HIDDEN = 128
RANK = 64

NC = 2
NS = 16
NW = NC * NS

B = 128
K = 80
EPAD = NW * K * B
NPAD = 10240
ROWS_PER_TILE = NPAD // NS
NPH = 2
K2 = K // NPH

W2 = 2 * RANK


def _prep_body(x_ref, w_ref, p_ref):
    feat = jnp.dot(x_ref[...], w_ref[...], preferred_element_type=jnp.float32)
    logp = jnp.log(jnp.maximum(jnp.abs(feat), 1e-30))
    gp = jnp.where(feat < 0, 3.0, 2.0)
    p_ref[...] = jnp.concatenate([logp, gp], axis=1)


def _finish_body(pp_ref, v_ref, o_ref):
    a = pp_ref[0] + pp_ref[1]
    s = a[:, :RANK]
    g = a[:, RANK:]
    sign = 1.0 - 2.0 * jnp.mod(g, 2.0)
    neigh = jnp.where(g > 0.0, sign * jnp.exp(s), 0.0)
    o_ref[...] = lax.dot_general(neigh, v_ref[...],
                                 (((1,), (1,)), ((), ())),
                                 preferred_element_type=jnp.float32)


def _sc_scatter_body(p_hbm, src_hbm, dst_hbm, out_hbm,
                     src_v, dst_v, gbuf, gbuf1, acc, sem, sem1):
    c = lax.axis_index("c")
    s = lax.axis_index("s")
    wid = s * NC + c
    gbufs = (gbuf, gbuf1)
    sems = (sem, sem1)

    zv = jnp.zeros((16,), jnp.float32)

    def zrow(i, carry):
        for l in range(W2 // 16):
            gbuf[i, pl.ds(l * 16, 16)] = zv
        return carry

    lax.fori_loop(0, B, zrow, 0)
    for r in range(ROWS_PER_TILE // B):
        pltpu.sync_copy(gbuf, acc.at[pl.ds(s * ROWS_PER_TILE + r * B, B)])

    plsc.subcore_barrier()

    for ph in range(NPH):
        pltpu.sync_copy(src_hbm.at[wid, pl.ds(ph * K2, K2)], src_v)
        pltpu.sync_copy(dst_hbm.at[wid, pl.ds(ph * K2, K2)], dst_v)
        pltpu.async_copy(p_hbm.at[src_v.at[0]], gbufs[0], sems[0])

        def body(t, carry):
            for b in range(2):
                j = t * 2 + b
                jn = lax.rem(j + 1, K2)
                pltpu.async_copy(p_hbm.at[src_v.at[jn]],
                                 gbufs[(b + 1) % 2], sems[(b + 1) % 2])
                pltpu.make_async_copy(p_hbm.at[src_v.at[0]], gbufs[b],
                                      sems[b]).wait()
                pltpu.sync_copy(gbufs[b], acc.at[dst_v.at[j]], add=True)
            return carry

        lax.fori_loop(0, K2 // 2, body, 0)
        pltpu.make_async_copy(p_hbm.at[src_v.at[0]], gbufs[0], sems[0]).wait()

    plsc.subcore_barrier()

    pltpu.sync_copy(acc.at[pl.ds(s * ROWS_PER_TILE, ROWS_PER_TILE)],
                    out_hbm.at[c, pl.ds(s * ROWS_PER_TILE, ROWS_PER_TILE)])


_sc_scatter = functools.partial(
    pl.kernel,
    out_type=jax.ShapeDtypeStruct((NC, NPAD, W2), jnp.float32),
    mesh=plsc.VectorSubcoreMesh(core_axis_name="c", subcore_axis_name="s"),
    scratch_types=[
        pltpu.VMEM((K2, B), jnp.int32),
        pltpu.VMEM((K2, B), jnp.int32),
        pltpu.VMEM((B, W2), jnp.float32),
        pltpu.VMEM((B, W2), jnp.float32),
        pltpu.VMEM_SHARED((NPAD, W2), jnp.float32),
        pltpu.SemaphoreType.DMA,
        pltpu.SemaphoreType.DMA,
    ],
)(_sc_scatter_body)


def kernel(x, edge_index, W, V):
    blk = 10000
    P = pl.pallas_call(
        _prep_body,
        grid=(N // blk,),
        in_specs=[
            pl.BlockSpec((blk, IN_FEA), lambda i: (i, 0)),
            pl.BlockSpec((IN_FEA, RANK), lambda i: (0, 0)),
        ],
        out_specs=pl.BlockSpec((blk, W2), lambda i: (i, 0)),
        out_shape=jax.ShapeDtypeStruct((N, W2), jnp.float32),
    )(x, W)

    pad = EPAD - E
    cyc = jnp.arange(pad, dtype=jnp.int32) % (NPAD - N)
    ei = jnp.concatenate(
        [edge_index, jnp.stack([cyc, N + cyc])], axis=1)
    src_r = ei[0].reshape(NW, K, B)
    dst_r = ei[1].reshape(NW, K, B)

    partials = _sc_scatter(P, src_r, dst_r)

    blk2 = 10000
    out = pl.pallas_call(
        _finish_body,
        grid=(N // blk2,),
        in_specs=[
            pl.BlockSpec((NC, blk2, W2), lambda i: (0, i, 0)),
            pl.BlockSpec((IN_FEA, RANK), lambda i: (0, 0)),
        ],
        out_specs=pl.BlockSpec((blk2, HIDDEN), lambda i: (i, 0)),
        out_shape=jax.ShapeDtypeStruct((N, HIDDEN), jnp.float32),
    )(partials, V)
    return out


# speedup vs baseline: 1.0219x; 1.0219x over previous
import functools

import jax
import jax.numpy as jnp
from jax import lax
from jax.experimental import pallas as pl
from jax.experimental.pallas import tpu as pltpu
from jax.experimental.pallas import tpu_sc as plsc

N = 10000
E = 320000
IN_FEA = 128
HIDDEN = 128
RANK = 64

NC = 2
NS = 16
NW = NC * NS

B = 128
K = 80
EPAD = NW * K * B
NPAD = 10240
ROWS_PER_TILE = NPAD // NS
NPH = 2
K2 = K // NPH

W2 = 2 * RANK


def _prep_body(x_ref, w_ref, p_ref):
    feat = jnp.dot(x_ref[...], w_ref[...], preferred_element_type=jnp.float32)
    logp = jnp.log(jnp.maximum(jnp.abs(feat), 1e-30))
    gp = jnp.where(feat < 0, 3.0, 2.0)
    p_ref[...] = jnp.concatenate([logp, gp], axis=1)


def _finish_body(pp_ref, v_ref, o_ref):
    a = pp_ref[0] + pp_ref[1]
    s = a[:, :RANK]
    g = a[:, RANK:]
    sign = 1.0 - 2.0 * jnp.mod(g, 2.0)
    neigh = jnp.where(g > 0.0, sign * jnp.exp(s), 0.0)
    o_ref[...] = lax.dot_general(neigh, v_ref[...],
                                 (((1,), (1,)), ((), ())),
                                 preferred_element_type=jnp.float32)


def _sc_scatter_body(p_hbm, src_hbm, dst_hbm, out_hbm,
                     src_v, dst_v, gbuf, gbuf1, acc, sem, sem1):
    c = lax.axis_index("c")
    s = lax.axis_index("s")
    wid = s * NC + c
    gbufs = (gbuf, gbuf1)
    sems = (sem, sem1)

    zv = jnp.zeros((16,), jnp.float32)

    def zrow(i, carry):
        for l in range(W2 // 16):
            gbuf[i, pl.ds(l * 16, 16)] = zv
        return carry

    lax.fori_loop(0, B, zrow, 0)
    for r in range(ROWS_PER_TILE // B):
        pltpu.sync_copy(gbuf, acc.at[pl.ds(s * ROWS_PER_TILE + r * B, B)])

    plsc.subcore_barrier()

    for ph in range(NPH):
        pltpu.sync_copy(src_hbm.at[wid, pl.ds(ph * K2, K2)], src_v)
        pltpu.sync_copy(dst_hbm.at[wid, pl.ds(ph * K2, K2)], dst_v)
        pltpu.async_copy(p_hbm.at[src_v.at[0]], gbufs[0], sems[0])

        def body(t, carry):
            for b in range(2):
                j = t * 2 + b
                jn = lax.rem(j + 1, K2)
                pltpu.async_copy(p_hbm.at[src_v.at[jn]],
                                 gbufs[(b + 1) % 2], sems[(b + 1) % 2])
                pltpu.make_async_copy(p_hbm.at[src_v.at[0]], gbufs[b],
                                      sems[b]).wait()
                pltpu.sync_copy(gbufs[b], acc.at[dst_v.at[j]], add=True)
            return carry

        lax.fori_loop(0, K2 // 2, body, 0)
        pltpu.make_async_copy(p_hbm.at[src_v.at[0]], gbufs[0], sems[0]).wait()

    plsc.subcore_barrier()

    pltpu.sync_copy(acc.at[pl.ds(s * ROWS_PER_TILE, ROWS_PER_TILE)],
                    out_hbm.at[c, pl.ds(s * ROWS_PER_TILE, ROWS_PER_TILE)])


_sc_scatter = functools.partial(
    pl.kernel,
    out_type=jax.ShapeDtypeStruct((NC, NPAD, W2), jnp.float32),
    mesh=plsc.VectorSubcoreMesh(core_axis_name="c", subcore_axis_name="s"),
    scratch_types=[
        pltpu.VMEM((K2, B), jnp.int32),
        pltpu.VMEM((K2, B), jnp.int32),
        pltpu.VMEM((B, W2), jnp.float32),
        pltpu.VMEM((B, W2), jnp.float32),
        pltpu.VMEM_SHARED((NPAD, W2), jnp.float32),
        pltpu.SemaphoreType.DMA,
        pltpu.SemaphoreType.DMA,
    ],
)(_sc_scatter_body)


def kernel(x, edge_index, W, V):
    blk = 5000
    P = pl.pallas_call(
        _prep_body,
        grid=(N // blk,),
        in_specs=[
            pl.BlockSpec((blk, IN_FEA), lambda i: (i, 0)),
            pl.BlockSpec((IN_FEA, RANK), lambda i: (0, 0)),
        ],
        out_specs=pl.BlockSpec((blk, W2), lambda i: (i, 0)),
        out_shape=jax.ShapeDtypeStruct((N, W2), jnp.float32),
    )(x, W)

    pad = EPAD - E
    cyc = jnp.arange(pad, dtype=jnp.int32) % (NPAD - N)
    ei = jnp.concatenate(
        [edge_index, jnp.stack([cyc, N + cyc])], axis=1)
    src_r = ei[0].reshape(NW, K, B)
    dst_r = ei[1].reshape(NW, K, B)

    partials = _sc_scatter(P, src_r, dst_r)

    blk2 = 5000
    out = pl.pallas_call(
        _finish_body,
        grid=(N // blk2,),
        in_specs=[
            pl.BlockSpec((NC, blk2, W2), lambda i: (0, i, 0)),
            pl.BlockSpec((IN_FEA, RANK), lambda i: (0, 0)),
        ],
        out_specs=pl.BlockSpec((blk2, HIDDEN), lambda i: (i, 0)),
        out_shape=jax.ShapeDtypeStruct((N, HIDDEN), jnp.float32),
    )(partials, V)
    return out
